# Initial kernel scaffold; baseline (speedup 1.0000x reference)
import jax
import jax.numpy as jnp
from jax.experimental import pallas as pl


def kernel(inputs, token_table, pos_table):
    raise NotImplementedError("write your pallas kernel here")


# speedup vs baseline: 1.4187x; 1.4187x over previous
import functools

import jax
import jax.numpy as jnp
from jax import lax
from jax.experimental import pallas as pl
from jax.experimental.pallas import tpu as pltpu
from jax.experimental.pallas import tpu_sc as plsc

NUM_CORES = 2
NUM_SUBCORES = 16
NUM_WORKERS = NUM_CORES * NUM_SUBCORES


@functools.cache
def _make_kernel(n_rows, seq_len, embed, chunk_rows):
  assert n_rows % (NUM_WORKERS * chunk_rows) == 0
  assert chunk_rows % seq_len == 0
  rows_per_worker = n_rows // NUM_WORKERS
  n_chunks = rows_per_worker // chunk_rows
  seqs_per_chunk = chunk_rows // seq_len

  mesh = plsc.VectorSubcoreMesh(
      core_axis_name="c", subcore_axis_name="s",
      num_cores=NUM_CORES, num_subcores=NUM_SUBCORES)

  @functools.partial(
      pl.kernel,
      out_type=jax.ShapeDtypeStruct((n_rows, embed), jnp.float32),
      mesh=mesh,
      compiler_params=pltpu.CompilerParams(use_tc_tiling_on_sc=False),
      scratch_types=[
          pltpu.VMEM((chunk_rows,), jnp.int32),
          pltpu.VMEM((chunk_rows, embed), jnp.float32),
          pltpu.VMEM_SHARED((seq_len, embed), jnp.float32),
          pltpu.SemaphoreType.DMA,
      ],
  )
  def k(table_hbm, idx_hbm, pos_hbm, out_hbm, idx_v, rows_v, pos_sh, sem):
    sid = lax.axis_index("s")
    wid = sid * NUM_CORES + lax.axis_index("c")
    base = wid * rows_per_worker

    @pl.when(sid == 0)
    def _():
      pltpu.sync_copy(pos_hbm, pos_sh)

    plsc.subcore_barrier()

    def body(c, _):
      off = pl.multiple_of(base + c * chunk_rows, chunk_rows)
      pltpu.sync_copy(idx_hbm.at[pl.ds(off, chunk_rows)], idx_v)
      for s in range(seqs_per_chunk):
        pltpu.sync_copy(pos_sh, rows_v.at[pl.ds(s * seq_len, seq_len)])
      pltpu.async_copy(table_hbm.at[idx_v], rows_v, sem, add=True).wait()
      pltpu.sync_copy(rows_v, out_hbm.at[pl.ds(off, chunk_rows)])
      return ()

    lax.fori_loop(0, n_chunks, body, (), unroll=False)

  return k


def kernel(inputs, token_table, pos_table):
  batch, seq_len = inputs.shape
  _, embed = token_table.shape
  n_rows = batch * seq_len
  idx = inputs.reshape(n_rows).astype(jnp.int32)
  k = _make_kernel(n_rows, seq_len, embed, chunk_rows=1600)
  out = k(token_table, idx, pos_table)
  return out.reshape(batch, seq_len, embed)
